# baseline (speedup 1.0000x reference)
import functools

import jax
import jax.numpy as jnp
from jax.experimental import pallas as pl
from jax.experimental.pallas import tpu as pltpu

B, S, D = 2, 3, 16
WMINOR = 384
BASE = 32
ACC_LO, ACC_HI = 32, 352


def _conv_body(w_ref, grid_ref, mask_ref, out_ref):
    i = pl.program_id(1)
    j = pl.program_id(2)

    @pl.when((i == 0) & (j == 0))
    def _init():
        out_ref[...] = jnp.zeros_like(out_ref)

    def tap_block(k, acc):
        t1 = k // 9
        t2 = (k // 3) % 3
        t3 = k % 3
        sub = grid_ref[0, i + t1, j + t2, pl.ds(t3, D), :, :]
        tbase = k * 27
        for t4 in range(3):
            for t5 in range(3):
                for t6 in range(3):
                    dlt = (t5 - 1) * 18 + (t6 - 1)
                    t = tbase + t4 * 9 + t5 * 3 + t6
                    wv = jnp.stack([w_ref[0, t], w_ref[1, t]]
                                   ).astype(jnp.bfloat16)
                    src = sub[:, t4:t4 + D, ACC_LO + dlt:ACC_HI + dlt]
                    acc = acc + wv[:, None, None, None] * src[None]
        return acc

    acc = jax.lax.fori_loop(
        0, 27, tap_block,
        jnp.zeros((2, D, D, ACC_HI - ACC_LO), jnp.bfloat16))
    m = mask_ref[0, 0, 0, :, :, ACC_LO:ACC_HI].astype(jnp.float32)
    s = (jax.nn.sigmoid(acc[0].astype(jnp.float32))
         + jax.nn.sigmoid(acc[1].astype(jnp.float32))) * m
    out_ref[0, :, :, ACC_LO:ACC_HI] += s


@functools.partial(jax.jit, static_argnames=())
def kernel(coords, values, w):
    feats = jax.nn.relu(values).astype(jnp.bfloat16)
    b = coords[:, 0]
    col = BASE + (coords[:, 5] + 1) * 18 + (coords[:, 6] + 1)
    grid = jnp.zeros((B, S + 2, S + 2, D + 2, D + 2, WMINOR), jnp.bfloat16)
    grid = grid.at[b, coords[:, 1] + 1, coords[:, 2] + 1,
                   coords[:, 3] + 1, coords[:, 4] + 1, col].set(feats)
    mask = jnp.zeros((B, S, S, D, D, WMINOR), jnp.bfloat16)
    mask = mask.at[b, coords[:, 1], coords[:, 2],
                   coords[:, 3], coords[:, 4], col].set(1.0)

    w2 = jnp.transpose(w, (1, 0, 4, 5, 2, 3))
    wpair = jnp.stack([w.reshape(-1), w2.reshape(-1)])

    out = pl.pallas_call(
        _conv_body,
        grid=(B, S, S),
        in_specs=[
            pl.BlockSpec((2, 729), lambda bb, ii, jj: (0, 0),
                         memory_space=pltpu.SMEM),
            pl.BlockSpec((1, S + 2, S + 2, D + 2, D + 2, WMINOR),
                         lambda bb, ii, jj: (bb, 0, 0, 0, 0, 0)),
            pl.BlockSpec((1, 1, 1, D, D, WMINOR),
                         lambda bb, ii, jj: (bb, ii, jj, 0, 0, 0)),
        ],
        out_specs=pl.BlockSpec((1, D, D, WMINOR),
                               lambda bb, ii, jj: (bb, 0, 0, 0)),
        out_shape=jax.ShapeDtypeStruct((B, D, D, WMINOR), jnp.float32),
    )(wpair, grid, mask)

    return out[..., 51:339].reshape(B, D, D, D, 18)[..., :D]


# speedup vs baseline: 1.1453x; 1.1453x over previous
import functools

import jax
import jax.numpy as jnp
from jax import lax
from jax.experimental import pallas as pl
from jax.experimental.pallas import tpu as pltpu
from jax.experimental.pallas import tpu_sc as plsc

B, S, D = 2, 3, 16
WMINOR = 384
BASE = 32
ACC_LO, ACC_HI = 32, 352


def _conv_body(w_ref, grid_ref, mask_ref, out_ref):
    i = pl.program_id(1)
    j = pl.program_id(2)

    @pl.when((i == 0) & (j == 0))
    def _init():
        out_ref[...] = jnp.zeros_like(out_ref)

    def tap_block(k, acc):
        t1 = k // 9
        t2 = (k // 3) % 3
        t3 = k % 3
        sub = grid_ref[0, i + t1, j + t2, pl.ds(t3, D), :, :]
        tbase = k * 27
        for t4 in range(3):
            for t5 in range(3):
                for t6 in range(3):
                    dlt = (t5 - 1) * 18 + (t6 - 1)
                    t = tbase + t4 * 9 + t5 * 3 + t6
                    wv = jnp.stack([w_ref[0, t], w_ref[1, t]])
                    src = sub[:, t4:t4 + D, ACC_LO + dlt:ACC_HI + dlt]
                    acc = acc + wv[:, None, None, None] * src[None]
        return acc

    acc = jax.lax.fori_loop(
        0, 27, tap_block,
        jnp.zeros((2, D, D, ACC_HI - ACC_LO), jnp.float32))
    m = mask_ref[0, 0, 0, :, :, ACC_LO:ACC_HI]
    s = (jax.nn.sigmoid(acc[0]) + jax.nn.sigmoid(acc[1])) * m
    out_ref[0, :, :, ACC_LO:ACC_HI] += s


N = 100000
NT = 16
PT = 6400
NP = NT * PT
NB = PT // 128
GSZ = B * (S + 2) * (S + 2) * (D + 2) * (D + 2) * WMINOR
MSZ = B * S * S * D * D * WMINOR


ZCH = 12960
ZCM = 6912


def _sc_scatter_body(coords_ref, values_ref,
                     grid_ref, mask_ref, cvm, vvm, idxb, valb, zbuf, sem):
    cid = lax.axis_index("c")
    sid = lax.axis_index("s")
    base = sid * PT

    def zfill(v, carry):
        zbuf[pl.ds(v * 16, 16)] = jnp.zeros((16,), jnp.float32)
        return carry

    lax.fori_loop(0, ZCH // 16, zfill, 0)

    @pl.when(cid == 0)
    def _zero_grid():
        def go(t, carry):
            pltpu.sync_copy(zbuf,
                            grid_ref.at[pl.ds(sid * (GSZ // NT) + t * ZCH,
                                              ZCH)])
            return carry
        lax.fori_loop(0, GSZ // NT // ZCH, go, 0)

    @pl.when(cid == 1)
    def _zero_mask():
        def go(t, carry):
            pltpu.sync_copy(zbuf.at[pl.ds(0, ZCM)],
                            mask_ref.at[pl.ds(sid * (MSZ // NT) + t * ZCM,
                                              ZCM)])
            return carry
        lax.fori_loop(0, MSZ // NT // ZCM, go, 0)

    plsc.subcore_barrier()

    pltpu.sync_copy(coords_ref.at[:, pl.ds(base, PT)], cvm)
    pltpu.sync_copy(values_ref.at[pl.ds(base, PT)], vvm)

    def compute(v, carry):
        o = v * 16
        c = [cvm[j, pl.ds(o, 16)] for j in range(7)]
        col = BASE + (c[5] + 1) * 18 + (c[6] + 1)
        grow = (((c[0] * 5 + c[1] + 1) * 5 + c[2] + 1) * 18
                + c[3] + 1) * 18 + c[4] + 1
        mrow = (((c[0] * 3 + c[1]) * 3 + c[2]) * 16 + c[3]) * 16 + c[4]
        idx = jnp.where(cid == 0, grow, mrow) * WMINOR + col
        val = jnp.where(cid == 0, jnp.maximum(vvm[pl.ds(o, 16)], 0.0), 1.0)
        valid = (base + o + lax.iota(jnp.int32, 16)) < N
        idx = jnp.where(valid, idx, 0)
        r = v // 8
        q = (v % 8) * 16
        idxb[r, pl.ds(q, 16)] = idx
        valb[r, pl.ds(q, 16)] = val
        return carry

    lax.fori_loop(0, PT // 16, compute, 0)

    @pl.when(cid == 0)
    def _scatter_grid():
        def go(j, carry):
            pltpu.async_copy(valb.at[j], grid_ref.at[idxb.at[j]], sem).wait()
            return carry
        lax.fori_loop(0, NB, go, 0)

    @pl.when(cid == 1)
    def _scatter_mask():
        def go(j, carry):
            pltpu.async_copy(valb.at[j], mask_ref.at[idxb.at[j]], sem).wait()
            return carry
        lax.fori_loop(0, NB, go, 0)


def _sc_scatter(coords, values):
    coords_p = jnp.pad(coords.T, ((0, 0), (0, NP - N)))
    values_p = jnp.pad(values, (0, NP - N))
    mesh = plsc.VectorSubcoreMesh(core_axis_name="c", subcore_axis_name="s")
    fn = functools.partial(
        pl.kernel, mesh=mesh,
        out_type=[jax.ShapeDtypeStruct((GSZ,), jnp.float32),
                  jax.ShapeDtypeStruct((MSZ,), jnp.float32)],
        scratch_types=[
            pltpu.VMEM((7, PT), jnp.int32),
            pltpu.VMEM((PT,), jnp.float32),
            pltpu.VMEM((NB, 128), jnp.int32),
            pltpu.VMEM((NB, 128), jnp.float32),
            pltpu.VMEM((ZCH,), jnp.float32),
            pltpu.SemaphoreType.DMA,
        ],
    )(_sc_scatter_body)
    grid_flat, mask_flat = fn(coords_p, values_p)
    return (grid_flat.reshape(B, S + 2, S + 2, D + 2, D + 2, WMINOR),
            mask_flat.reshape(B, S, S, D, D, WMINOR))


@functools.partial(jax.jit, static_argnames=())
def kernel(coords, values, w):
    grid, mask = _sc_scatter(coords, values)

    w2 = jnp.transpose(w, (1, 0, 4, 5, 2, 3))
    wpair = jnp.stack([w.reshape(-1), w2.reshape(-1)])

    out = pl.pallas_call(
        _conv_body,
        grid=(B, S, S),
        in_specs=[
            pl.BlockSpec((2, 729), lambda bb, ii, jj: (0, 0),
                         memory_space=pltpu.SMEM),
            pl.BlockSpec((1, S + 2, S + 2, D + 2, D + 2, WMINOR),
                         lambda bb, ii, jj: (bb, 0, 0, 0, 0, 0)),
            pl.BlockSpec((1, 1, 1, D, D, WMINOR),
                         lambda bb, ii, jj: (bb, ii, jj, 0, 0, 0)),
        ],
        out_specs=pl.BlockSpec((1, D, D, WMINOR),
                               lambda bb, ii, jj: (bb, 0, 0, 0)),
        out_shape=jax.ShapeDtypeStruct((B, D, D, WMINOR), jnp.float32),
    )(wpair, grid, mask)

    return out[..., 51:339].reshape(B, D, D, D, 18)[..., :D]


# speedup vs baseline: 3.5842x; 3.1295x over previous
import functools

import jax
import jax.numpy as jnp
from jax import lax
from jax.experimental import pallas as pl
from jax.experimental.pallas import tpu as pltpu
from jax.experimental.pallas import tpu_sc as plsc

B, S, D = 2, 3, 16
WMINOR = 384
BASE = 32
ACC_LO, ACC_HI = 32, 352


ACC_W = ACC_HI - ACC_LO


def _conv_body(band_ref, grid_ref, mask_ref, out_ref):
    i = pl.program_id(1)
    j = pl.program_id(2)

    @pl.when((i == 0) & (j == 0))
    def _init():
        out_ref[...] = jnp.zeros_like(out_ref)

    def tap_block(k, acc):
        t1 = k // 9
        t2 = (k // 3) % 3
        t3 = k % 3
        sub = grid_ref[0, i + t1, j + t2, pl.ds(t3, D), :, :]
        for t4 in range(3):
            a = sub[:, t4:t4 + D, :].reshape(D * D, WMINOR)
            bnd = band_ref[k * 3 + t4]
            acc = acc + jnp.dot(a, bnd, preferred_element_type=jnp.float32)
        return acc

    acc = jax.lax.fori_loop(
        0, 27, tap_block,
        jnp.zeros((D * D, 2 * ACC_W), jnp.float32))
    m = mask_ref[0, 0, 0].reshape(D * D, WMINOR)[:, ACC_LO:ACC_HI]
    s = (jax.nn.sigmoid(acc[:, :ACC_W]) + jax.nn.sigmoid(acc[:, ACC_W:])) * m
    out_ref[0, :, :, ACC_LO:ACC_HI] += s.reshape(D, D, ACC_W)


N = 100000
NT = 16
PT = 6400
NP = NT * PT
NB = PT // 128
GSZ = B * (S + 2) * (S + 2) * (D + 2) * (D + 2) * WMINOR
MSZ = B * S * S * D * D * WMINOR


ZCH = 12960
ZCM = 6912


def _sc_scatter_body(coords_ref, values_ref,
                     grid_ref, mask_ref, cvm, vvm, idxb, valb, zbuf, sem):
    cid = lax.axis_index("c")
    sid = lax.axis_index("s")
    base = sid * PT

    def zfill(v, carry):
        zbuf[pl.ds(v * 16, 16)] = jnp.zeros((16,), jnp.float32)
        return carry

    lax.fori_loop(0, ZCH // 16, zfill, 0)

    @pl.when(cid == 0)
    def _zero_grid():
        def go(t, carry):
            pltpu.sync_copy(zbuf,
                            grid_ref.at[pl.ds(sid * (GSZ // NT) + t * ZCH,
                                              ZCH)])
            return carry
        lax.fori_loop(0, GSZ // NT // ZCH, go, 0)

    @pl.when(cid == 1)
    def _zero_mask():
        def go(t, carry):
            pltpu.sync_copy(zbuf.at[pl.ds(0, ZCM)],
                            mask_ref.at[pl.ds(sid * (MSZ // NT) + t * ZCM,
                                              ZCM)])
            return carry
        lax.fori_loop(0, MSZ // NT // ZCM, go, 0)

    plsc.subcore_barrier()

    pltpu.sync_copy(coords_ref.at[:, pl.ds(base, PT)], cvm)
    pltpu.sync_copy(values_ref.at[pl.ds(base, PT)], vvm)

    def compute(v, carry):
        o = v * 16
        c = [cvm[j, pl.ds(o, 16)] for j in range(7)]
        col = BASE + (c[5] + 1) * 18 + (c[6] + 1)
        grow = (((c[0] * 5 + c[1] + 1) * 5 + c[2] + 1) * 18
                + c[3] + 1) * 18 + c[4] + 1
        mrow = (((c[0] * 3 + c[1]) * 3 + c[2]) * 16 + c[3]) * 16 + c[4]
        idx = jnp.where(cid == 0, grow, mrow) * WMINOR + col
        val = jnp.where(cid == 0, jnp.maximum(vvm[pl.ds(o, 16)], 0.0), 1.0)
        valid = (base + o + lax.iota(jnp.int32, 16)) < N
        idx = jnp.where(valid, idx, 0)
        r = v // 8
        q = (v % 8) * 16
        idxb[r, pl.ds(q, 16)] = idx
        valb[r, pl.ds(q, 16)] = val
        return carry

    lax.fori_loop(0, PT // 16, compute, 0)

    @pl.when(cid == 0)
    def _scatter_grid():
        def go(j, carry):
            pltpu.async_copy(valb.at[j], grid_ref.at[idxb.at[j]], sem).wait()
            return carry
        lax.fori_loop(0, NB, go, 0)

    @pl.when(cid == 1)
    def _scatter_mask():
        def go(j, carry):
            pltpu.async_copy(valb.at[j], mask_ref.at[idxb.at[j]], sem).wait()
            return carry
        lax.fori_loop(0, NB, go, 0)


def _sc_scatter(coords, values):
    coords_p = jnp.pad(coords.T, ((0, 0), (0, NP - N)))
    values_p = jnp.pad(values, (0, NP - N))
    mesh = plsc.VectorSubcoreMesh(core_axis_name="c", subcore_axis_name="s")
    fn = functools.partial(
        pl.kernel, mesh=mesh,
        out_type=[jax.ShapeDtypeStruct((GSZ,), jnp.float32),
                  jax.ShapeDtypeStruct((MSZ,), jnp.float32)],
        scratch_types=[
            pltpu.VMEM((7, PT), jnp.int32),
            pltpu.VMEM((PT,), jnp.float32),
            pltpu.VMEM((NB, 128), jnp.int32),
            pltpu.VMEM((NB, 128), jnp.float32),
            pltpu.VMEM((ZCH,), jnp.float32),
            pltpu.SemaphoreType.DMA,
        ],
    )(_sc_scatter_body)
    grid_flat, mask_flat = fn(coords_p, values_p)
    return (grid_flat.reshape(B, S + 2, S + 2, D + 2, D + 2, WMINOR),
            mask_flat.reshape(B, S, S, D, D, WMINOR))


@functools.partial(jax.jit, static_argnames=())
def kernel(coords, values, w):
    grid, mask = _sc_scatter(coords, values)
    grid = grid.astype(jnp.bfloat16)

    w2 = jnp.transpose(w, (1, 0, 4, 5, 2, 3))
    eye9 = jnp.stack([jnp.eye(WMINOR, k=-(18 * (t5 - 1) + (t6 - 1)),
                              dtype=jnp.float32)[:, ACC_LO:ACC_HI]
                      for t5 in range(3) for t6 in range(3)])
    band = jnp.concatenate(
        [jnp.tensordot(ww.reshape(81, 9), eye9, axes=[[1], [0]])
         for ww in (w, w2)], axis=2).astype(jnp.bfloat16)

    out = pl.pallas_call(
        _conv_body,
        grid=(B, S, S),
        in_specs=[
            pl.BlockSpec((81, WMINOR, 2 * ACC_W),
                         lambda bb, ii, jj: (0, 0, 0)),
            pl.BlockSpec((1, S + 2, S + 2, D + 2, D + 2, WMINOR),
                         lambda bb, ii, jj: (bb, 0, 0, 0, 0, 0)),
            pl.BlockSpec((1, 1, 1, D, D, WMINOR),
                         lambda bb, ii, jj: (bb, ii, jj, 0, 0, 0)),
        ],
        out_specs=pl.BlockSpec((1, D, D, WMINOR),
                               lambda bb, ii, jj: (bb, 0, 0, 0)),
        out_shape=jax.ShapeDtypeStruct((B, D, D, WMINOR), jnp.float32),
    )(band, grid, mask)

    return out[..., 51:339].reshape(B, D, D, D, 18)[..., :D]


# speedup vs baseline: 3.6343x; 1.0140x over previous
import functools

import jax
import jax.numpy as jnp
from jax import lax
from jax.experimental import pallas as pl
from jax.experimental.pallas import tpu as pltpu
from jax.experimental.pallas import tpu_sc as plsc

B, S, D = 2, 3, 16
WMINOR = 384
BASE = 32
ACC_LO, ACC_HI = 32, 352


ACC_W = ACC_HI - ACC_LO


def _conv_body(band_ref, grid_ref, mask_ref, out_ref):
    i = pl.program_id(1)
    j = pl.program_id(2)

    @pl.when((i == 0) & (j == 0))
    def _init():
        out_ref[...] = jnp.zeros_like(out_ref)

    def tap_block(k, acc):
        t1 = k // 9
        t2 = (k // 3) % 3
        t3 = k % 3
        sub = grid_ref[0, i + t1, j + t2, pl.ds(t3, D), :, :]
        for t4 in range(3):
            a = sub[:, t4:t4 + D, :].reshape(D * D, WMINOR)
            bnd = band_ref[k * 3 + t4]
            acc = acc + jnp.dot(a, bnd, preferred_element_type=jnp.float32)
        return acc

    acc = jax.lax.fori_loop(
        0, 27, tap_block,
        jnp.zeros((D * D, 2 * ACC_W), jnp.float32))
    m = mask_ref[0, 0, 0].reshape(D * D, WMINOR)[:, ACC_LO:ACC_HI]
    s = (jax.nn.sigmoid(acc[:, :ACC_W]) + jax.nn.sigmoid(acc[:, ACC_W:])) * m
    out_ref[0, :, :, ACC_LO:ACC_HI] += s.reshape(D, D, ACC_W)


N = 100000
NT = 16
PT = 6400
NP = NT * PT
NB = PT // 128
GSZ = B * (S + 2) * (S + 2) * (D + 2) * (D + 2) * WMINOR
MSZ = B * S * S * D * D * WMINOR


ZCH = 12960
ZCM = 6912
RND = 10


def _sc_scatter_body(gidx_ref, vals_ref, midx_ref,
                     grid_ref, mask_ref, idxb, valb, zbuf, sem):
    cid = lax.axis_index("c")
    sid = lax.axis_index("s")

    def zfill(v, carry):
        zbuf[pl.ds(v * 16, 16)] = jnp.zeros((16,), jnp.float32)
        return carry

    lax.fori_loop(0, ZCH // 16, zfill, 0)

    @pl.when(cid == 0)
    def _zero_grid():
        cps = [pltpu.async_copy(
                   zbuf, grid_ref.at[pl.ds(sid * (GSZ // NT) + t * ZCH, ZCH)],
                   sem)
               for t in range(GSZ // NT // ZCH)]
        for c in cps:
            c.wait()

    @pl.when(cid == 1)
    def _zero_mask():
        cps = [pltpu.async_copy(
                   zbuf.at[pl.ds(0, ZCM)],
                   mask_ref.at[pl.ds(sid * (MSZ // NT) + t * ZCM, ZCM)],
                   sem)
               for t in range(MSZ // NT // ZCM)]
        for c in cps:
            c.wait()

    plsc.subcore_barrier()

    @pl.when(cid == 0)
    def _scatter_grid():
        pltpu.sync_copy(gidx_ref.at[sid], idxb)
        pltpu.sync_copy(vals_ref.at[sid], valb)

        def go(r, carry):
            cps = [pltpu.async_copy(valb.at[r * RND + u],
                                    grid_ref.at[idxb.at[r * RND + u]], sem)
                   for u in range(RND)]
            for c in cps:
                c.wait()
            return carry
        lax.fori_loop(0, NB // RND, go, 0)

    @pl.when(cid == 1)
    def _scatter_mask():
        pltpu.sync_copy(midx_ref.at[sid], idxb)

        def ofill(v, carry):
            valb[0, pl.ds(v * 16, 16)] = jnp.ones((16,), jnp.float32)
            return carry

        lax.fori_loop(0, 8, ofill, 0)

        def go(r, carry):
            cps = [pltpu.async_copy(valb.at[0],
                                    mask_ref.at[idxb.at[r * RND + u]], sem)
                   for u in range(RND)]
            for c in cps:
                c.wait()
            return carry
        lax.fori_loop(0, NB // RND, go, 0)


def _sc_scatter(coords, values):
    col = BASE + (coords[:, 5] + 1) * 18 + (coords[:, 6] + 1)
    grow = (((coords[:, 0] * 5 + coords[:, 1] + 1) * 5 + coords[:, 2] + 1)
            * 18 + coords[:, 3] + 1) * 18 + coords[:, 4] + 1
    mrow = (((coords[:, 0] * 3 + coords[:, 1]) * 3 + coords[:, 2]) * 16
            + coords[:, 3]) * 16 + coords[:, 4]
    gidx = jnp.pad(grow * WMINOR + col, (0, NP - N)).reshape(NT, NB, 128)
    midx = jnp.pad(mrow * WMINOR + col, (0, NP - N)).reshape(NT, NB, 128)
    vals = jnp.pad(jax.nn.relu(values), (0, NP - N)).reshape(NT, NB, 128)

    mesh = plsc.VectorSubcoreMesh(core_axis_name="c", subcore_axis_name="s")
    fn = functools.partial(
        pl.kernel, mesh=mesh,
        out_type=[jax.ShapeDtypeStruct((GSZ,), jnp.float32),
                  jax.ShapeDtypeStruct((MSZ,), jnp.float32)],
        scratch_types=[
            pltpu.VMEM((NB, 128), jnp.int32),
            pltpu.VMEM((NB, 128), jnp.float32),
            pltpu.VMEM((ZCH,), jnp.float32),
            pltpu.SemaphoreType.DMA,
        ],
    )(_sc_scatter_body)
    grid_flat, mask_flat = fn(gidx, vals, midx)
    return (grid_flat.reshape(B, S + 2, S + 2, D + 2, D + 2, WMINOR),
            mask_flat.reshape(B, S, S, D, D, WMINOR))


@functools.partial(jax.jit, static_argnames=())
def kernel(coords, values, w):
    grid, mask = _sc_scatter(coords, values)
    grid = grid.astype(jnp.bfloat16)

    w2 = jnp.transpose(w, (1, 0, 4, 5, 2, 3))
    eye9 = jnp.stack([jnp.eye(WMINOR, k=-(18 * (t5 - 1) + (t6 - 1)),
                              dtype=jnp.float32)[:, ACC_LO:ACC_HI]
                      for t5 in range(3) for t6 in range(3)])
    band = jnp.concatenate(
        [jnp.tensordot(ww.reshape(81, 9), eye9, axes=[[1], [0]])
         for ww in (w, w2)], axis=2).astype(jnp.bfloat16)

    out = pl.pallas_call(
        _conv_body,
        grid=(B, S, S),
        in_specs=[
            pl.BlockSpec((81, WMINOR, 2 * ACC_W),
                         lambda bb, ii, jj: (0, 0, 0)),
            pl.BlockSpec((1, S + 2, S + 2, D + 2, D + 2, WMINOR),
                         lambda bb, ii, jj: (bb, 0, 0, 0, 0, 0)),
            pl.BlockSpec((1, 1, 1, D, D, WMINOR),
                         lambda bb, ii, jj: (bb, ii, jj, 0, 0, 0)),
        ],
        out_specs=pl.BlockSpec((1, D, D, WMINOR),
                               lambda bb, ii, jj: (bb, 0, 0, 0)),
        out_shape=jax.ShapeDtypeStruct((B, D, D, WMINOR), jnp.float32),
    )(band, grid, mask)

    return out[..., 51:339].reshape(B, D, D, D, 18)[..., :D]
